# Initial kernel scaffold; baseline (speedup 1.0000x reference)
import jax
import jax.numpy as jnp
from jax.experimental import pallas as pl


def kernel(cls_embeddings, user_indices, is_word_indices, user_emb, gamma, beta, Wv1, bv1, Wv2, bv2, Wa1, ba1, Wa2, ba2):
    raise NotImplementedError("write your pallas kernel here")


# speedup vs baseline: 1.2418x; 1.2418x over previous
import functools

import jax
import jax.numpy as jnp
from jax import lax
from jax.experimental import pallas as pl
from jax.experimental.pallas import tpu as pltpu
from jax.experimental.pallas import tpu_sc as plsc

_NC = 2
_NS = 16
_CHUNK = 128


def _gather_rows(user_emb, user_indices):
    B = user_indices.shape[0]
    E = user_emb.shape[1]
    nw = _NC * _NS
    bpw = B // nw
    nch = bpw // _CHUNK
    idx2d = user_indices.astype(jnp.int32).reshape(nw * nch, _CHUNK)
    mesh = plsc.VectorSubcoreMesh(core_axis_name="c", subcore_axis_name="s")

    @functools.partial(
        pl.kernel,
        mesh=mesh,
        out_type=jax.ShapeDtypeStruct((B, E), jnp.float32),
        compiler_params=pltpu.CompilerParams(use_tc_tiling_on_sc=False),
        scratch_types=[
            pltpu.VMEM((nch, _CHUNK), jnp.int32),
            pltpu.VMEM((nch, _CHUNK, E), jnp.float32),
            pltpu.SemaphoreType.DMA,
        ],
    )
    def gk(table_hbm, idx_hbm, out_hbm, idx_v, rows_v, sem):
        wid = lax.axis_index("s") * _NC + lax.axis_index("c")
        pltpu.sync_copy(idx_hbm.at[pl.ds(wid * nch, nch)], idx_v)
        copies = [
            pltpu.async_copy(table_hbm.at[idx_v.at[c]], rows_v.at[c], sem)
            for c in range(nch)
        ]
        for cp in copies:
            cp.wait()
        for c in range(nch):
            pltpu.sync_copy(
                rows_v.at[c], out_hbm.at[pl.ds(wid * bpw + c * _CHUNK, _CHUNK)]
            )

    return gk(user_emb, idx2d)


def _dense_body(cls_ref, u_ref, w_ref, g1c_ref, ws_ref, b2_ref, v_ref, a_ref,
                *, d_in):
    cls = cls_ref[...]
    u = u_ref[...]
    w = w_ref[...]
    s = jnp.sum(cls, axis=1, keepdims=True) + jnp.sum(u, axis=1, keepdims=True) + w
    q = (jnp.sum(cls * cls, axis=1, keepdims=True)
         + jnp.sum(u * u, axis=1, keepdims=True) + w * w)
    mu = s * (1.0 / d_in)
    var = q * (1.0 / d_in) - mu * mu
    inv = lax.rsqrt(var + 1e-5)

    t = jnp.dot(cls, g1c_ref[...], preferred_element_type=jnp.float32)
    ws = ws_ref[...]
    t = (t + u[:, 0:1] * ws[0:1] + u[:, 1:2] * ws[1:2]
         + u[:, 2:3] * ws[2:3] + u[:, 3:4] * ws[3:4] + w * ws[4:5])
    y = (t - mu * ws[5:6]) * inv + ws[6:7]
    h = y * 0.5 * (1.0 + lax.erf(y * 0.7071067811865476))
    zv = jnp.sum(h * ws[7:8], axis=1) + b2_ref[0]
    za = jnp.sum(h * ws[8:9], axis=1) + b2_ref[1]
    v_ref[...] = jax.nn.sigmoid(zv)
    a_ref[...] = jax.nn.sigmoid(za)


def kernel(cls_embeddings, user_indices, is_word_indices, user_emb, gamma, beta,
           Wv1, bv1, Wv2, bv2, Wa1, ba1, Wa2, ba2):
    B, Dc = cls_embeddings.shape
    E = user_emb.shape[1]
    H = Wv1.shape[1]
    d_in = Dc + E + 1
    n2 = 2 * H
    NP = ((n2 + 127) // 128) * 128
    f32 = jnp.float32

    W1cat = jnp.concatenate([Wv1, Wa1], axis=1)
    b1cat = jnp.concatenate([bv1, ba1])
    G1 = gamma[:, None] * W1cat
    G1p = jnp.zeros((d_in, NP), f32).at[:, :n2].set(G1)
    g1c = G1p[:Dc]
    g1sum = jnp.sum(G1p, axis=0)
    c1 = jnp.zeros((NP,), f32).at[:n2].set(beta @ W1cat + b1cat)
    w2v = jnp.zeros((NP,), f32).at[:H].set(Wv2[:, 0])
    w2a = jnp.zeros((NP,), f32).at[H:n2].set(Wa2[:, 0])
    wsmall = jnp.zeros((16, NP), f32)
    wsmall = wsmall.at[0:E].set(G1p[Dc:Dc + E])
    wsmall = wsmall.at[4].set(G1p[Dc + E])
    wsmall = wsmall.at[5].set(g1sum)
    wsmall = wsmall.at[6].set(c1)
    wsmall = wsmall.at[7].set(w2v)
    wsmall = wsmall.at[8].set(w2a)
    b2 = jnp.concatenate([bv2, ba2]).astype(f32)

    u = _gather_rows(user_emb, user_indices)
    w2d = is_word_indices[:, None]

    bB = 512
    grid = (B // bB,)
    body = functools.partial(_dense_body, d_in=float(d_in))
    v, a = pl.pallas_call(
        body,
        grid=grid,
        in_specs=[
            pl.BlockSpec((bB, Dc), lambda i: (i, 0)),
            pl.BlockSpec((bB, E), lambda i: (i, 0)),
            pl.BlockSpec((bB, 1), lambda i: (i, 0)),
            pl.BlockSpec((Dc, NP), lambda i: (0, 0)),
            pl.BlockSpec((16, NP), lambda i: (0, 0)),
            pl.BlockSpec(memory_space=pltpu.SMEM),
        ],
        out_specs=[
            pl.BlockSpec((bB,), lambda i: (i,)),
            pl.BlockSpec((bB,), lambda i: (i,)),
        ],
        out_shape=[jax.ShapeDtypeStruct((B,), f32)] * 2,
    )(cls_embeddings, u, w2d, g1c, wsmall, b2)
    return (v, a)


# speedup vs baseline: 2.4585x; 1.9797x over previous
import functools

import jax
import jax.numpy as jnp
from jax import lax
from jax.experimental import pallas as pl
from jax.experimental.pallas import tpu as pltpu
from jax.experimental.pallas import tpu_sc as plsc

_NC = 2
_NS = 16
_CHUNK = 128


def _gather_features(cols, is_word, user_indices):
    B = user_indices.shape[0]
    E = len(cols)
    nw = _NC * _NS
    bpw = B // nw
    nch = bpw // _CHUNK
    idx1d = user_indices.astype(jnp.int32)
    mesh = plsc.VectorSubcoreMesh(core_axis_name="c", subcore_axis_name="s")

    @functools.partial(
        pl.kernel,
        mesh=mesh,
        out_type=jax.ShapeDtypeStruct((8, B), jnp.float32),
        compiler_params=pltpu.CompilerParams(use_tc_tiling_on_sc=False),
    scratch_types=[
            pltpu.VMEM((nch, _CHUNK), jnp.int32),
            pltpu.VMEM((nch, E + 1, _CHUNK), jnp.float32),
            pltpu.SemaphoreType.DMA,
            pltpu.SemaphoreType.DMA,
        ],
    )
    def gk(t0, t1, t2, t3, isw_hbm, idx_hbm, out_hbm, idx_v, vals_v, sem,
           sem2):
        tables = (t0, t1, t2, t3)
        wid = lax.axis_index("s") * _NC + lax.axis_index("c")
        base = wid * bpw
        copies = [pltpu.async_copy(
            idx_hbm.at[pl.ds(base + c * _CHUNK, _CHUNK)], idx_v.at[c], sem)
            for c in range(nch)]
        copies += [pltpu.async_copy(
            isw_hbm.at[pl.ds(base + c * _CHUNK, _CHUNK)], vals_v.at[c].at[E],
            sem) for c in range(nch)]
        for cp in copies[:nch]:
            cp.wait()
        gathers = [pltpu.async_copy(
            tables[e].at[idx_v.at[c]], vals_v.at[c].at[e], sem2)
            for c in range(nch) for e in range(E)]
        for cp in copies[nch:]:
            cp.wait()
        for cp in gathers:
            cp.wait()
        stores = [pltpu.async_copy(
            vals_v.at[c].at[e],
            out_hbm.at[e].at[pl.ds(base + c * _CHUNK, _CHUNK)], sem)
            for c in range(nch) for e in range(E + 1)]
        for cp in stores:
            cp.wait()

    return gk(*cols, is_word, idx1d)


def _dense_body(cls_ref, uw_ref, g1aug_ref, e0_ref, w10_ref, ws_ref, w2_ref,
                b2_ref, v_ref, a_ref, *, d_in, np_):
    cls = cls_ref[...]
    uw = uw_ref[0:5, :]
    t1 = jnp.dot(cls.astype(jnp.bfloat16), g1aug_ref[...],
                 preferred_element_type=jnp.float32)
    t2 = jnp.dot(cls * cls, e0_ref[...], preferred_element_type=jnp.float32)
    lhs = jnp.concatenate([uw, uw * uw], axis=0)
    u = lax.dot_general(lhs, w10_ref[...], (((0,), (0,)), ((), ())),
                        preferred_element_type=jnp.float32)
    s = t1[:, np_:np_ + 1] + u[:, np_:np_ + 1]
    q = t2[:, 0:1] + u[:, np_ + 1:np_ + 2]
    mu = s * (1.0 / d_in)
    var = q * (1.0 / d_in) - mu * mu
    inv = lax.rsqrt(var + 1e-5)
    ws = ws_ref[...]
    y = (t1[:, :np_] + u[:, :np_] - mu * ws[0:1]) * inv + ws[1:2]
    h = y * 0.5 * (1.0 + lax.erf(y * 0.7071067811865476))
    zt = lax.dot_general(w2_ref[...], h, (((0,), (1,)), ((), ())),
                         preferred_element_type=jnp.float32)
    v_ref[...] = jax.nn.sigmoid(zt[0] + b2_ref[0])
    a_ref[...] = jax.nn.sigmoid(zt[1] + b2_ref[1])


def kernel(cls_embeddings, user_indices, is_word_indices, user_emb, gamma, beta,
           Wv1, bv1, Wv2, bv2, Wa1, ba1, Wa2, ba2):
    B, Dc = cls_embeddings.shape
    E = user_emb.shape[1]
    H = Wv1.shape[1]
    d_in = Dc + E + 1
    n2 = 2 * H
    NP = ((n2 + 127) // 128) * 128
    NA = NP + 128
    f32 = jnp.float32

    cols = [user_emb[:, e] for e in range(E)]
    uw8 = _gather_features(cols, is_word_indices, user_indices)

    W1cat = jnp.concatenate([Wv1, Wa1], axis=1)
    b1cat = jnp.concatenate([bv1, ba1])
    G1 = gamma[:, None] * W1cat
    g1sum = jnp.zeros((NP,), f32).at[:n2].set(jnp.sum(G1, axis=0))
    c1 = jnp.zeros((NP,), f32).at[:n2].set(beta @ W1cat + b1cat)
    w2v = jnp.zeros((NP,), f32).at[:H].set(Wv2[:, 0])
    w2a = jnp.zeros((NP,), f32).at[H:n2].set(Wa2[:, 0])
    g1aug = jnp.zeros((Dc, NA), f32).at[:, :n2].set(G1[:Dc])
    g1aug = g1aug.at[:, NP].set(1.0)
    e0 = jnp.zeros((Dc, 128), f32).at[:, 0].set(1.0)
    w10 = jnp.zeros((16, NA), f32)
    w10 = w10.at[0:E + 1, :n2].set(G1[Dc:])
    w10 = w10.at[0:E + 1, NP].set(1.0)
    w10 = w10.at[5:10, NP + 1].set(1.0)
    w10 = w10[:10]
    wsmall = jnp.stack([g1sum, c1] + [jnp.zeros((NP,), f32)] * 6)
    w2pair = jnp.zeros((NP, 128), f32).at[:, 0].set(w2v).at[:, 1].set(w2a)
    b2 = jnp.concatenate([bv2, ba2]).astype(f32)

    bB = 4096
    grid = (B // bB,)
    body = functools.partial(_dense_body, d_in=float(d_in), np_=NP)
    v, a = pl.pallas_call(
        body,
        grid=grid,
        in_specs=[
            pl.BlockSpec((bB, Dc), lambda i: (i, 0)),
            pl.BlockSpec((8, bB), lambda i: (0, i)),
            pl.BlockSpec((Dc, NA), lambda i: (0, 0)),
            pl.BlockSpec((Dc, 128), lambda i: (0, 0)),
            pl.BlockSpec((10, NA), lambda i: (0, 0)),
            pl.BlockSpec((8, NP), lambda i: (0, 0)),
            pl.BlockSpec((NP, 128), lambda i: (0, 0)),
            pl.BlockSpec(memory_space=pltpu.SMEM),
        ],
        out_specs=[
            pl.BlockSpec((bB,), lambda i: (i,)),
            pl.BlockSpec((bB,), lambda i: (i,)),
        ],
        out_shape=[jax.ShapeDtypeStruct((B,), f32)] * 2,
    )(cls_embeddings, uw8, g1aug.astype(jnp.bfloat16), e0, w10, wsmall, w2pair, b2)
    return (v, a)
